# baseline (speedup 1.0000x reference)
import functools

import jax
import jax.numpy as jnp
from jax import lax
from jax.experimental import pallas as pl
from jax.experimental.pallas import tpu as pltpu
from jax.experimental.pallas import tpu_sc as plsc

NVOCAB = 100000
NHID = 64
MAXLEN = 200
BATCH = 4096
SEQ = 200

NUM_CORES = 2
NUM_SUBCORES = 16
NW = NUM_CORES * NUM_SUBCORES
CH = 2
ROWS_PER_CH = CH * SEQ
LANES = 16
NBUF = 4
LA_G = 2
LA_I = 3
NPIECE = 4

_mesh = plsc.VectorSubcoreMesh(core_axis_name="c", subcore_axis_name="s")


def _make_embed(nbatch):
    seq_per_w = nbatch // NW
    nchunk = seq_per_w // CH

    @functools.partial(
        pl.kernel,
        out_type=jax.ShapeDtypeStruct((nbatch * SEQ, NHID), jnp.float32),
        mesh=_mesh,
        scratch_types=[
            pltpu.VMEM((SEQ, NHID), jnp.float32),
            [pltpu.VMEM((ROWS_PER_CH, NHID), jnp.float32) for _ in range(NBUF)],
            [pltpu.VMEM((ROWS_PER_CH,), jnp.int32) for _ in range(NBUF)],
            [pltpu.SemaphoreType.DMA for _ in range(NBUF)],
            [pltpu.SemaphoreType.DMA for _ in range(NBUF)],
            [pltpu.SemaphoreType.DMA for _ in range(NBUF)],
        ],
        compiler_params=pltpu.CompilerParams(use_tc_tiling_on_sc=False),
    )
    def _embed(x_hbm, emb_hbm, pos_hbm, out_hbm, pos_v, rows, idxs, gsem, ssem, isem):
        wid = lax.axis_index("s") * NUM_CORES + lax.axis_index("c")
        base = wid * (seq_per_w * SEQ)

        pltpu.sync_copy(pos_hbm, pos_v)

        def idx_desc(g, b):
            return pltpu.make_async_copy(
                x_hbm.at[pl.ds(base + g * ROWS_PER_CH, ROWS_PER_CH)], idxs[b], isem[b])

        def gather_desc(g, b):
            return pltpu.make_async_copy(emb_hbm.at[idxs[b]], rows[b], gsem[b])

        def store_desc(g, b):
            return pltpu.make_async_copy(
                rows[b], out_hbm.at[pl.ds(base + g * ROWS_PER_CH, ROWS_PER_CH)], ssem[b])

        def chunk_body(g, j, issue_idx, wait_store, issue_gather):
            if issue_idx:
                idx_desc(g + LA_I, (j + LA_I) % NBUF).start()
            if wait_store:
                store_desc(g - (NBUF - LA_G), (j + LA_G) % NBUF).wait()
            if issue_gather:
                idx_desc(g + LA_G, (j + LA_G) % NBUF).wait()
                gather_desc(g + LA_G, (j + LA_G) % NBUF).start()

            gather_desc(g, j).wait()

            def add_rows(r, c2, _rows=rows[j]):
                for c in range(NHID // LANES):
                    sl = pl.ds(LANES * c, LANES)
                    p = pos_v[r, sl]
                    _rows[r, sl] += p
                    _rows[SEQ + r, sl] += p
                return c2

            lax.fori_loop(0, SEQ, add_rows, 0, unroll=4)
            store_desc(g, j).start()

        for n in range(LA_I):
            idx_desc(n, n % NBUF).start()
        for n in range(LA_G):
            idx_desc(n, n % NBUF).wait()
            gather_desc(n, n % NBUF).start()

        for g in range(NBUF):
            chunk_body(g, g, g + LA_I < nchunk, g >= NBUF - LA_G, g + LA_G < nchunk)

        nsteady = (nchunk - LA_I) // NBUF * NBUF

        def step(t, carry):
            for j in range(NBUF):
                chunk_body(t * NBUF + j, j, True, True, True)
            return carry

        lax.fori_loop(1, nsteady // NBUF, step, 0)

        for g in range(nsteady, nchunk):
            chunk_body(g, g % NBUF, g + LA_I < nchunk, True, g + LA_G < nchunk)

        for k in range(NBUF - LA_G):
            g = nchunk - (NBUF - LA_G) + k
            store_desc(g, g % NBUF).wait()

    return _embed


_embed_piece = _make_embed(BATCH // NPIECE)


def kernel(x, emb_table, pos_table):
    nb = BATCH // NPIECE
    out = jnp.zeros((BATCH, SEQ, NHID), jnp.float32)
    for k in range(NPIECE):
        xk = x[k * nb:(k + 1) * nb].reshape(-1).astype(jnp.int32)
        ok = _embed_piece(xk, emb_table, pos_table)
        out = lax.dynamic_update_slice(out, ok.reshape(nb, SEQ, NHID), (k * nb, 0, 0))
    return out


# speedup vs baseline: 1.0479x; 1.0479x over previous
import functools

import jax
import jax.numpy as jnp
from jax import lax
from jax.experimental import pallas as pl
from jax.experimental.pallas import tpu as pltpu
from jax.experimental.pallas import tpu_sc as plsc

NVOCAB = 100000
NHID = 64
MAXLEN = 200
BATCH = 4096
SEQ = 200

NUM_CORES = 2
NUM_SUBCORES = 16
NW = NUM_CORES * NUM_SUBCORES
CH = 2
ROWS_PER_CH = CH * SEQ
LANES = 16
NBUF = 4
LA_G = 2
LA_I = 3
NPIECE = 4

_mesh = plsc.VectorSubcoreMesh(core_axis_name="c", subcore_axis_name="s")


def _make_embed(nbatch):
    seq_per_w = nbatch // NW
    nchunk = seq_per_w // CH

    @functools.partial(
        pl.kernel,
        out_type=jax.ShapeDtypeStruct((nbatch * SEQ, NHID), jnp.float32),
        mesh=_mesh,
        scratch_types=[
            pltpu.VMEM((SEQ, NHID), jnp.float32),
            [pltpu.VMEM((ROWS_PER_CH, NHID), jnp.float32) for _ in range(NBUF)],
            [pltpu.VMEM((ROWS_PER_CH,), jnp.int32) for _ in range(NBUF)],
            [pltpu.SemaphoreType.DMA for _ in range(NBUF)],
            [pltpu.SemaphoreType.DMA for _ in range(NBUF)],
            [pltpu.SemaphoreType.DMA for _ in range(NBUF)],
        ],
        compiler_params=pltpu.CompilerParams(use_tc_tiling_on_sc=False),
    )
    def _embed(x_hbm, emb_hbm, pos_hbm, out_hbm, pos_v, rows, idxs, gsem, ssem, isem):
        wid = lax.axis_index("s") * NUM_CORES + lax.axis_index("c")
        base = wid * (seq_per_w * SEQ)

        pltpu.sync_copy(pos_hbm, pos_v)

        def idx_desc(g, b):
            return pltpu.make_async_copy(
                x_hbm.at[pl.ds(base + g * ROWS_PER_CH, ROWS_PER_CH)], idxs[b], isem[b])

        def gather_desc(g, b):
            return pltpu.make_async_copy(emb_hbm.at[idxs[b]], rows[b], gsem[b])

        def store_desc(g, b):
            return pltpu.make_async_copy(
                rows[b], out_hbm.at[pl.ds(base + g * ROWS_PER_CH, ROWS_PER_CH)], ssem[b])

        def chunk_body(g, j, issue_idx, wait_store, issue_gather):
            if issue_idx:
                idx_desc(g + LA_I, (j + LA_I) % NBUF).start()
            if wait_store:
                store_desc(g - (NBUF - LA_G), (j + LA_G) % NBUF).wait()
            if issue_gather:
                idx_desc(g + LA_G, (j + LA_G) % NBUF).wait()
                gather_desc(g + LA_G, (j + LA_G) % NBUF).start()

            gather_desc(g, j).wait()

            def add_rows(r, c2, _rows=rows[j]):
                for c in range(NHID // LANES):
                    sl = pl.ds(LANES * c, LANES)
                    p = pos_v[r, sl]
                    _rows[r, sl] += p
                    _rows[SEQ + r, sl] += p
                return c2

            lax.fori_loop(0, SEQ, add_rows, 0, unroll=4)
            store_desc(g, j).start()

        for n in range(LA_I):
            idx_desc(n, n % NBUF).start()
        for n in range(LA_G):
            idx_desc(n, n % NBUF).wait()
            gather_desc(n, n % NBUF).start()

        for g in range(NBUF):
            chunk_body(g, g, g + LA_I < nchunk, g >= NBUF - LA_G, g + LA_G < nchunk)

        nsteady = (nchunk - LA_I) // NBUF * NBUF

        def step(t, carry):
            for j in range(NBUF):
                chunk_body(t * NBUF + j, j, True, True, True)
            return carry

        lax.fori_loop(1, nsteady // NBUF, step, 0)

        for g in range(nsteady, nchunk):
            chunk_body(g, g % NBUF, g + LA_I < nchunk, True, g + LA_G < nchunk)

        for k in range(NBUF - LA_G):
            g = nchunk - (NBUF - LA_G) + k
            store_desc(g, g % NBUF).wait()

    return _embed


_embed_piece = _make_embed(BATCH // NPIECE)


def kernel(x, emb_table, pos_table):
    nb = BATCH // NPIECE
    pieces = []
    for k in range(NPIECE):
        xk = x[k * nb:(k + 1) * nb].reshape(-1).astype(jnp.int32)
        ok = _embed_piece(xk, emb_table, pos_table)
        pieces.append(ok.reshape(nb, SEQ, NHID))
    return jnp.concatenate(pieces, axis=0)
